# baseline (speedup 1.0000x reference)
import functools

import jax
import jax.numpy as jnp
from jax import lax
from jax.experimental import pallas as pl
from jax.experimental.pallas import tpu as pltpu
from jax.experimental.pallas import tpu_sc as plsc

KNN = 4


def _knn_y_kernel(xt_ref, x_ref, wt_ref, bias_ref,
                  y0, y1, y2, y3, i0, i1, i2, i3):
    a = xt_ref[0]
    xm = x_ref[0]
    tr = a.shape[0]
    n = xm.shape[1]

    dot = lax.dot_general(a, xm, (((1,), (0,)), ((), ())),
                          preferred_element_type=jnp.float32)
    sq_r = jnp.sum(a * a, axis=1, keepdims=True)
    sq_c = jnp.sum(xm * xm, axis=0, keepdims=True)
    d = jnp.sqrt(jnp.maximum(sq_r + sq_c - 2.0 * dot, 0.0))

    iota = lax.broadcasted_iota(jnp.int32, (tr, n), 1)
    gbase = pl.program_id(0) * n
    idx_refs = (i0, i1, i2, i3)
    y_refs = (y0, y1, y2, y3)
    for k in range(KNN):
        m = jnp.min(d, axis=1, keepdims=True)
        ik = jnp.min(jnp.where(d == m, iota, n), axis=1, keepdims=True)
        idx_refs[k][0] = ik + gbase
        d = jnp.where(iota == ik, jnp.inf, d)
        yk = lax.dot_general(a, wt_ref[k], (((1,), (0,)), ((), ())),
                             preferred_element_type=jnp.float32)
        if k == 0:
            yk = yk + bias_ref[...]
        y_refs[k][0] = yk


def _tc_stage(xt, x, wt, bias2, tr):
    B, N, C = xt.shape
    O = wt.shape[2]
    return pl.pallas_call(
        _knn_y_kernel,
        grid=(B, N // tr),
        in_specs=[
            pl.BlockSpec((1, tr, C), lambda b, t: (b, t, 0)),
            pl.BlockSpec((1, C, N), lambda b, t: (b, 0, 0)),
            pl.BlockSpec((KNN, C, O), lambda b, t: (0, 0, 0)),
            pl.BlockSpec((1, O), lambda b, t: (0, 0)),
        ],
        out_specs=(
            [pl.BlockSpec((1, tr, O), lambda b, t: (b, t, 0))] * KNN
            + [pl.BlockSpec((1, tr, 1), lambda b, t: (b, t, 0))] * KNN
        ),
        out_shape=(
            [jax.ShapeDtypeStruct((B, N, O), jnp.float32)] * KNN
            + [jax.ShapeDtypeStruct((B, N, 1), jnp.int32)] * KNN
        ),
    )(xt, x, wt, bias2)


def _make_sc_gather_sum(bn, o, ch):
    info = plsc.get_sparse_core_info()
    nc, ns = info.num_cores, info.num_subcores
    nw = nc * ns
    rows_per_w = bn // nw
    n_chunks = rows_per_w // ch
    mesh = plsc.VectorSubcoreMesh(core_axis_name="c", subcore_axis_name="s")

    @functools.partial(
        pl.kernel,
        out_type=jax.ShapeDtypeStruct((bn, o), jnp.float32),
        mesh=mesh,
        scratch_types=(
            [pltpu.VMEM((rows_per_w,), jnp.int32) for _ in range(KNN)]
            + [pltpu.VMEM((ch, o), jnp.float32) for _ in range(2 * KNN)]
            + [pltpu.VMEM((ch, o), jnp.float32) for _ in range(2)]
            + [pltpu.SemaphoreType.DMA for _ in range(2)]
            + [pltpu.SemaphoreType.DMA for _ in range(2)]
        ),
    )
    def sc_gather_sum(y0, y1, y2, y3, i0, i1, i2, i3, out,
                      ib0, ib1, ib2, ib3,
                      ga0, ga1, ga2, ga3, gb0, gb1, gb2, gb3,
                      oba, obb, sg0, sg1, so0, so1):
        wid = lax.axis_index("s") * nc + lax.axis_index("c")
        base0 = wid * rows_per_w
        ys = (y0, y1, y2, y3)
        idx_hbm = (i0, i1, i2, i3)
        ibs = (ib0, ib1, ib2, ib3)
        gsets = ((ga0, ga1, ga2, ga3), (gb0, gb1, gb2, gb3))
        obufs = (oba, obb)
        gsems = (sg0, sg1)
        osems = (so0, so1)

        for k in range(KNN):
            pltpu.sync_copy(idx_hbm[k].at[pl.ds(base0, rows_per_w)], ibs[k])

        def fire(g):
            p = g % 2
            return [pltpu.async_copy(
                        ys[k].at[ibs[k].at[pl.ds(g * ch, ch)]],
                        gsets[p][k], gsems[p])
                    for k in range(KNN)]

        gath = {0: fire(0)}
        ocopies = {}
        for g in range(n_chunks):
            p = g % 2
            if g + 1 < n_chunks:
                gath[g + 1] = fire(g + 1)
            for c in gath.pop(g):
                c.wait()
            if g >= 2:
                for c in ocopies.pop(g - 2):
                    c.wait()
            gbufs = gsets[p]
            ob = obufs[p]

            def row_body(j, carry):
                for c16 in range(o // 16):
                    sl = pl.ds(c16 * 16, 16)
                    ob[j, sl] = (gbufs[0][j, sl] + gbufs[1][j, sl]
                                 + gbufs[2][j, sl] + gbufs[3][j, sl])
                return carry

            lax.fori_loop(0, ch, row_body, 0)
            ocopies[g] = [pltpu.async_copy(
                ob, out.at[pl.ds(base0 + g * ch, ch)], osems[p])]
        for g in list(ocopies):
            for c in ocopies.pop(g):
                c.wait()

    return sc_gather_sum


def kernel(x, W, b):
    B, C, N = x.shape
    O = W.shape[0]
    xt = x.transpose(0, 2, 1)
    wt = W.transpose(2, 1, 0)
    bias2 = b.reshape(1, O)

    hb = B // 2
    sc = _make_sc_gather_sum(hb * N, O, ch=32)
    halves = []
    for h in range(2):
        xh = x[h * hb:(h + 1) * hb]
        xth = xt[h * hb:(h + 1) * hb]
        outs = _tc_stage(xth, xh, wt, bias2, tr=256)
        ys = [o.reshape(hb * N, O) for o in outs[:KNN]]
        idxs = [o.reshape(hb * N) for o in outs[KNN:]]
        halves.append(sc(*ys, *idxs).reshape(hb, N, O))
    out_t = jnp.concatenate(halves, axis=0)
    return out_t.transpose(0, 2, 1)


# speedup vs baseline: 1.1010x; 1.1010x over previous
import functools

import jax
import jax.numpy as jnp
from jax import lax
from jax.experimental import pallas as pl
from jax.experimental.pallas import tpu as pltpu
from jax.experimental.pallas import tpu_sc as plsc

KNN = 4


def _knn_y_kernel(xcol_ref, x_ref, wt_ref, bias_ref,
                  y0, y1, y2, y3, i0, i1, i2, i3):
    a = jnp.swapaxes(xcol_ref[0], 0, 1)
    xm = x_ref[0]
    tr = a.shape[0]
    n = xm.shape[1]

    dot = lax.dot_general(a, xm, (((1,), (0,)), ((), ())),
                          preferred_element_type=jnp.float32)
    sq_r = jnp.sum(a * a, axis=1, keepdims=True)
    sq_c = jnp.sum(xm * xm, axis=0, keepdims=True)
    d = jnp.sqrt(jnp.maximum(sq_r + sq_c - 2.0 * dot, 0.0))

    iota = lax.broadcasted_iota(jnp.int32, (tr, n), 1)
    gbase = pl.program_id(0) * n
    idx_refs = (i0, i1, i2, i3)
    y_refs = (y0, y1, y2, y3)
    for k in range(KNN):
        m = jnp.min(d, axis=1, keepdims=True)
        ik = jnp.min(jnp.where(d == m, iota, n), axis=1, keepdims=True)
        idx_refs[k][0] = ik + gbase
        d = jnp.where(iota == ik, jnp.inf, d)
        yk = lax.dot_general(a, wt_ref[k], (((1,), (0,)), ((), ())),
                             preferred_element_type=jnp.float32)
        if k == 0:
            yk = yk + bias_ref[...]
        y_refs[k][0] = yk


def _tc_stage(x, wt, bias2, tr):
    B, C, N = x.shape
    O = wt.shape[2]
    return pl.pallas_call(
        _knn_y_kernel,
        grid=(B, N // tr),
        in_specs=[
            pl.BlockSpec((1, C, tr), lambda b, t: (b, 0, t)),
            pl.BlockSpec((1, C, N), lambda b, t: (b, 0, 0)),
            pl.BlockSpec((KNN, C, O), lambda b, t: (0, 0, 0)),
            pl.BlockSpec((1, O), lambda b, t: (0, 0)),
        ],
        out_specs=(
            [pl.BlockSpec((1, tr, O), lambda b, t: (b, t, 0))] * KNN
            + [pl.BlockSpec((1, tr, 1), lambda b, t: (b, t, 0))] * KNN
        ),
        out_shape=(
            [jax.ShapeDtypeStruct((B, N, O), jnp.float32)] * KNN
            + [jax.ShapeDtypeStruct((B, N, 1), jnp.int32)] * KNN
        ),
    )(x, x, wt, bias2)


def _make_sc_gather_sum(bn, o, ch):
    info = plsc.get_sparse_core_info()
    nc, ns = info.num_cores, info.num_subcores
    nw = nc * ns
    rows_per_w = bn // nw
    n_chunks = rows_per_w // ch
    mesh = plsc.VectorSubcoreMesh(core_axis_name="c", subcore_axis_name="s")

    @functools.partial(
        pl.kernel,
        out_type=jax.ShapeDtypeStruct((bn, o), jnp.float32),
        mesh=mesh,
        scratch_types=(
            [pltpu.VMEM((rows_per_w,), jnp.int32) for _ in range(KNN)]
            + [pltpu.VMEM((ch, o), jnp.float32) for _ in range(2 * KNN)]
            + [pltpu.VMEM((ch, o), jnp.float32) for _ in range(2)]
            + [pltpu.SemaphoreType.DMA for _ in range(2)]
            + [pltpu.SemaphoreType.DMA for _ in range(2)]
        ),
    )
    def sc_gather_sum(y0, y1, y2, y3, i0, i1, i2, i3, out,
                      ib0, ib1, ib2, ib3,
                      ga0, ga1, ga2, ga3, gb0, gb1, gb2, gb3,
                      oba, obb, sg0, sg1, so0, so1):
        wid = lax.axis_index("s") * nc + lax.axis_index("c")
        base0 = wid * rows_per_w
        ys = (y0, y1, y2, y3)
        idx_hbm = (i0, i1, i2, i3)
        ibs = (ib0, ib1, ib2, ib3)
        gsets = ((ga0, ga1, ga2, ga3), (gb0, gb1, gb2, gb3))
        obufs = (oba, obb)
        gsems = (sg0, sg1)
        osems = (so0, so1)

        for k in range(KNN):
            pltpu.sync_copy(idx_hbm[k].at[pl.ds(base0, rows_per_w)], ibs[k])

        def fire(g):
            p = g % 2
            return [pltpu.async_copy(
                        ys[k].at[ibs[k].at[pl.ds(g * ch, ch)]],
                        gsets[p][k], gsems[p])
                    for k in range(KNN)]

        gath = {0: fire(0)}
        ocopies = {}
        for g in range(n_chunks):
            p = g % 2
            if g + 1 < n_chunks:
                gath[g + 1] = fire(g + 1)
            for c in gath.pop(g):
                c.wait()
            if g >= 2:
                for c in ocopies.pop(g - 2):
                    c.wait()
            gbufs = gsets[p]
            ob = obufs[p]

            def row_body(j, carry):
                for c16 in range(o // 16):
                    sl = pl.ds(c16 * 16, 16)
                    ob[j, sl] = (gbufs[0][j, sl] + gbufs[1][j, sl]
                                 + gbufs[2][j, sl] + gbufs[3][j, sl])
                return carry

            lax.fori_loop(0, ch, row_body, 0)
            ocopies[g] = [pltpu.async_copy(
                ob, out.at[pl.ds(base0 + g * ch, ch)], osems[p])]
        for g in list(ocopies):
            for c in ocopies.pop(g):
                c.wait()

    return sc_gather_sum


def kernel(x, W, b):
    B, C, N = x.shape
    O = W.shape[0]
    wt = W.transpose(2, 1, 0)
    bias2 = b.reshape(1, O)

    outs = _tc_stage(x, wt, bias2, tr=256)
    ys = [o.reshape(B * N, O) for o in outs[:KNN]]
    idxs = [o.reshape(B * N) for o in outs[KNN:]]

    sc = _make_sc_gather_sum(B * N, O, ch=32)
    out_t = sc(*ys, *idxs)
    return out_t.reshape(B, N, O).transpose(0, 2, 1)


# speedup vs baseline: 1.1853x; 1.0766x over previous
import functools

import jax
import jax.numpy as jnp
from jax import lax
from jax.experimental import pallas as pl
from jax.experimental.pallas import tpu as pltpu
from jax.experimental.pallas import tpu_sc as plsc

KNN = 4


def _knn_y_kernel(xcol_ref, x_ref, wt_ref, bias_ref,
                  y0, y1, y2, y3, i0, i1, i2, i3):
    a = jnp.swapaxes(xcol_ref[0], 0, 1)
    xm = x_ref[0]
    tr = a.shape[0]
    n = xm.shape[1]

    dot = lax.dot_general(a, xm, (((1,), (0,)), ((), ())),
                          preferred_element_type=jnp.float32)
    sq_r = jnp.sum(a * a, axis=1, keepdims=True)
    sq_c = jnp.sum(xm * xm, axis=0, keepdims=True)
    d = jnp.sqrt(jnp.maximum(sq_r + sq_c - 2.0 * dot, 0.0))

    iota = lax.broadcasted_iota(jnp.int32, (tr, n), 1)
    gbase = pl.program_id(0) * n
    idx_refs = (i0, i1, i2, i3)
    y_refs = (y0, y1, y2, y3)
    for k in range(KNN):
        m = jnp.min(d, axis=1, keepdims=True)
        ik = jnp.min(jnp.where(d == m, iota, n), axis=1, keepdims=True)
        idx_refs[k][0] = ik + gbase
        d = jnp.where(iota == ik, jnp.inf, d)
        yk = lax.dot_general(a, wt_ref[k], (((1,), (0,)), ((), ())),
                             preferred_element_type=jnp.float32)
        if k == 0:
            yk = yk + bias_ref[...]
        y_refs[k][0] = yk


def _tc_stage(x, wt, bias2, tr):
    B, C, N = x.shape
    O = wt.shape[2]
    return pl.pallas_call(
        _knn_y_kernel,
        grid=(B, N // tr),
        in_specs=[
            pl.BlockSpec((1, C, tr), lambda b, t: (b, 0, t)),
            pl.BlockSpec((1, C, N), lambda b, t: (b, 0, 0)),
            pl.BlockSpec((KNN, C, O), lambda b, t: (0, 0, 0)),
            pl.BlockSpec((1, O), lambda b, t: (0, 0)),
        ],
        out_specs=(
            [pl.BlockSpec((1, tr, O), lambda b, t: (b, t, 0))] * KNN
            + [pl.BlockSpec((1, tr, 1), lambda b, t: (b, t, 0))] * KNN
        ),
        out_shape=(
            [jax.ShapeDtypeStruct((B, N, O), jnp.float32)] * KNN
            + [jax.ShapeDtypeStruct((B, N, 1), jnp.int32)] * KNN
        ),
    )(x, x, wt, bias2)


def _make_sc_gather_sum(bn, o, ch):
    info = plsc.get_sparse_core_info()
    nc, ns = info.num_cores, info.num_subcores
    nw = nc * ns
    rows_per_w = bn // nw
    n_chunks = rows_per_w // ch
    mesh = plsc.VectorSubcoreMesh(core_axis_name="c", subcore_axis_name="s")

    @functools.partial(
        pl.kernel,
        out_type=jax.ShapeDtypeStruct((bn, o), jnp.float32),
        mesh=mesh,
        scratch_types=(
            [pltpu.VMEM((rows_per_w,), jnp.int32) for _ in range(KNN)]
            + [pltpu.VMEM((ch, o), jnp.float32) for _ in range(2 * KNN)]
            + [pltpu.VMEM((ch, o), jnp.float32) for _ in range(2)]
            + [pltpu.SemaphoreType.DMA for _ in range(2)]
            + [pltpu.SemaphoreType.DMA for _ in range(2)]
        ),
    )
    def sc_gather_sum(y0, y1, y2, y3, i0, i1, i2, i3, out,
                      ib0, ib1, ib2, ib3,
                      ga0, ga1, ga2, ga3, gb0, gb1, gb2, gb3,
                      oba, obb, sg0, sg1, so0, so1):
        wid = lax.axis_index("s") * nc + lax.axis_index("c")
        base0 = wid * rows_per_w
        ys = (y0, y1, y2, y3)
        idx_hbm = (i0, i1, i2, i3)
        ibs = (ib0, ib1, ib2, ib3)
        gsets = ((ga0, ga1, ga2, ga3), (gb0, gb1, gb2, gb3))
        obufs = (oba, obb)
        gsems = (sg0, sg1)
        osems = (so0, so1)

        for k in range(KNN):
            pltpu.sync_copy(idx_hbm[k].at[pl.ds(base0, rows_per_w)], ibs[k])

        def fire(g):
            p = g % 2
            return [pltpu.async_copy(
                        ys[k].at[ibs[k].at[pl.ds(g * ch, ch)]],
                        gsets[p][k], gsems[p])
                    for k in range(KNN)]

        gath = {0: fire(0)}
        ocopies = {}
        for g in range(n_chunks):
            p = g % 2
            if g + 1 < n_chunks:
                gath[g + 1] = fire(g + 1)
            for c in gath.pop(g):
                c.wait()
            if g >= 2:
                for c in ocopies.pop(g - 2):
                    c.wait()
            gbufs = gsets[p]
            ob = obufs[p]

            def row_body(j, carry):
                for c16 in range(o // 16):
                    sl = pl.ds(c16 * 16, 16)
                    ob[j, sl] = (gbufs[0][j, sl] + gbufs[1][j, sl]
                                 + gbufs[2][j, sl] + gbufs[3][j, sl])
                return carry

            lax.fori_loop(0, ch, row_body, 0)
            ocopies[g] = [pltpu.async_copy(
                ob, out.at[pl.ds(base0 + g * ch, ch)], osems[p])]
        for g in list(ocopies):
            for c in ocopies.pop(g):
                c.wait()

    return sc_gather_sum


def kernel(x, W, b):
    B, C, N = x.shape
    O = W.shape[0]
    wt = W.transpose(2, 1, 0)
    bias2 = b.reshape(1, O)

    outs = _tc_stage(x, wt, bias2, tr=512)
    ys = [o.reshape(B * N, O) for o in outs[:KNN]]
    idxs = [o.reshape(B * N) for o in outs[KNN:]]

    sc = _make_sc_gather_sum(B * N, O, ch=32)
    out_t = sc(*ys, *idxs)
    return out_t.reshape(B, N, O).transpose(0, 2, 1)


# speedup vs baseline: 1.2259x; 1.0342x over previous
import functools

import jax
import jax.numpy as jnp
from jax import lax
from jax.experimental import pallas as pl
from jax.experimental.pallas import tpu as pltpu
from jax.experimental.pallas import tpu_sc as plsc

KNN = 4


def _knn_y_kernel(xcol_ref, x_ref, wt_ref, bias_ref,
                  y0, y1, y2, y3, i0, i1, i2, i3):
    a = jnp.swapaxes(xcol_ref[0], 0, 1)
    xm = x_ref[0]
    tr = a.shape[0]
    n = xm.shape[1]

    dot = lax.dot_general(a, xm, (((1,), (0,)), ((), ())),
                          preferred_element_type=jnp.float32)
    sq_r = jnp.sum(a * a, axis=1, keepdims=True)
    sq_c = jnp.sum(xm * xm, axis=0, keepdims=True)
    d = jnp.sqrt(jnp.maximum(sq_r + sq_c - 2.0 * dot, 0.0))

    iota = lax.broadcasted_iota(jnp.int32, (tr, n), 1)
    gbase = pl.program_id(0) * n
    idx_refs = (i0, i1, i2, i3)
    y_refs = (y0, y1, y2, y3)
    for k in range(KNN):
        m = jnp.min(d, axis=1, keepdims=True)
        ik = jnp.min(jnp.where(d == m, iota, n), axis=1, keepdims=True)
        idx_refs[k][0] = ik + gbase
        d = jnp.where(iota == ik, jnp.inf, d)
        yk = lax.dot_general(a, wt_ref[k], (((1,), (0,)), ((), ())),
                             preferred_element_type=jnp.float32)
        if k == 0:
            yk = yk + bias_ref[...]
        y_refs[k][0] = yk


def _tc_stage(x, wt, bias2, tr):
    B, C, N = x.shape
    O = wt.shape[2]
    return pl.pallas_call(
        _knn_y_kernel,
        grid=(B, N // tr),
        in_specs=[
            pl.BlockSpec((1, C, tr), lambda b, t: (b, 0, t)),
            pl.BlockSpec((1, C, N), lambda b, t: (b, 0, 0)),
            pl.BlockSpec((KNN, C, O), lambda b, t: (0, 0, 0)),
            pl.BlockSpec((1, O), lambda b, t: (0, 0)),
        ],
        out_specs=(
            [pl.BlockSpec((1, tr, O), lambda b, t: (b, t, 0))] * KNN
            + [pl.BlockSpec((1, tr, 1), lambda b, t: (b, t, 0))] * KNN
        ),
        out_shape=(
            [jax.ShapeDtypeStruct((B, N, O), jnp.float32)] * KNN
            + [jax.ShapeDtypeStruct((B, N, 1), jnp.int32)] * KNN
        ),
    )(x, x, wt, bias2)


def _make_sc_gather_sum(bn, o, ch):
    info = plsc.get_sparse_core_info()
    nc, ns = info.num_cores, info.num_subcores
    nw = nc * ns
    rows_per_w = bn // nw
    n_chunks = rows_per_w // ch
    mesh = plsc.VectorSubcoreMesh(core_axis_name="c", subcore_axis_name="s")

    @functools.partial(
        pl.kernel,
        out_type=jax.ShapeDtypeStruct((bn, o), jnp.float32),
        mesh=mesh,
        scratch_types=(
            [pltpu.VMEM((rows_per_w,), jnp.int32) for _ in range(KNN)]
            + [pltpu.VMEM((ch, o), jnp.float32) for _ in range(2 * KNN)]
            + [pltpu.VMEM((ch, o), jnp.float32) for _ in range(2)]
            + [pltpu.SemaphoreType.DMA for _ in range(2)]
            + [pltpu.SemaphoreType.DMA for _ in range(2)]
        ),
    )
    def sc_gather_sum(y0, y1, y2, y3, i0, i1, i2, i3, out,
                      ib0, ib1, ib2, ib3,
                      ga0, ga1, ga2, ga3, gb0, gb1, gb2, gb3,
                      oba, obb, sg0, sg1, so0, so1):
        wid = lax.axis_index("s") * nc + lax.axis_index("c")
        base0 = wid * rows_per_w
        ys = (y0, y1, y2, y3)
        idx_hbm = (i0, i1, i2, i3)
        ibs = (ib0, ib1, ib2, ib3)
        gsets = ((ga0, ga1, ga2, ga3), (gb0, gb1, gb2, gb3))
        obufs = (oba, obb)
        gsems = (sg0, sg1)
        osems = (so0, so1)

        for k in range(KNN):
            pltpu.sync_copy(idx_hbm[k].at[pl.ds(base0, rows_per_w)], ibs[k])

        def fire(g):
            p = g % 2
            return [pltpu.async_copy(
                        ys[k].at[ibs[k].at[pl.ds(g * ch, ch)]],
                        gsets[p][k], gsems[p])
                    for k in range(KNN)]

        gath = {0: fire(0)}
        ocopies = {}
        for g in range(n_chunks):
            p = g % 2
            if g + 1 < n_chunks:
                gath[g + 1] = fire(g + 1)
            for c in gath.pop(g):
                c.wait()
            if g >= 2:
                for c in ocopies.pop(g - 2):
                    c.wait()
            gbufs = gsets[p]
            ob = obufs[p]

            def row_body(j, carry):
                for c16 in range(o // 16):
                    sl = pl.ds(c16 * 16, 16)
                    ob[j, sl] = (gbufs[0][j, sl] + gbufs[1][j, sl]
                                 + gbufs[2][j, sl] + gbufs[3][j, sl])
                return carry

            lax.fori_loop(0, ch, row_body, 0)
            ocopies[g] = [pltpu.async_copy(
                ob, out.at[pl.ds(base0 + g * ch, ch)], osems[p])]
        for g in list(ocopies):
            for c in ocopies.pop(g):
                c.wait()

    return sc_gather_sum


def kernel(x, W, b):
    B, C, N = x.shape
    O = W.shape[0]
    wt = W.transpose(2, 1, 0)
    bias2 = b.reshape(1, O)

    outs = _tc_stage(x, wt, bias2, tr=1024)
    ys = [o.reshape(B * N, O) for o in outs[:KNN]]
    idxs = [o.reshape(B * N) for o in outs[KNN:]]

    sc = _make_sc_gather_sum(B * N, O, ch=32)
    out_t = sc(*ys, *idxs)
    return out_t.reshape(B, N, O).transpose(0, 2, 1)


# speedup vs baseline: 1.2269x; 1.0008x over previous
import functools

import jax
import jax.numpy as jnp
from jax import lax
from jax.experimental import pallas as pl
from jax.experimental.pallas import tpu as pltpu
from jax.experimental.pallas import tpu_sc as plsc

KNN = 4


def _knn_y_kernel(xcol_ref, x_ref, wt_ref, bias_ref,
                  y0, y1, y2, y3, i0, i1, i2, i3):
    a = jnp.swapaxes(xcol_ref[0], 0, 1)
    xm = x_ref[0]
    tr = a.shape[0]
    n = xm.shape[1]

    dot = lax.dot_general(a, xm, (((1,), (0,)), ((), ())),
                          preferred_element_type=jnp.float32)
    sq_r = jnp.sum(a * a, axis=1, keepdims=True)
    sq_c = jnp.sum(xm * xm, axis=0, keepdims=True)
    d = jnp.sqrt(jnp.maximum(sq_r + sq_c - 2.0 * dot, 0.0))

    iota = lax.broadcasted_iota(jnp.int32, (tr, n), 1)
    gbase = pl.program_id(0) * n
    idx_refs = (i0, i1, i2, i3)
    y_refs = (y0, y1, y2, y3)
    for k in range(KNN):
        m = jnp.min(d, axis=1, keepdims=True)
        ik = jnp.min(jnp.where(d == m, iota, n), axis=1, keepdims=True)
        idx_refs[k][0] = ik + gbase
        if k + 1 < KNN:
            d = jnp.where(iota == ik, jnp.inf, d)
        yk = lax.dot_general(a, wt_ref[k], (((1,), (0,)), ((), ())),
                             preferred_element_type=jnp.float32)
        if k == 0:
            yk = yk + bias_ref[...]
        y_refs[k][0] = yk


def _tc_stage(x, wt, bias2, tr):
    B, C, N = x.shape
    O = wt.shape[2]
    return pl.pallas_call(
        _knn_y_kernel,
        grid=(B, N // tr),
        in_specs=[
            pl.BlockSpec((1, C, tr), lambda b, t: (b, 0, t)),
            pl.BlockSpec((1, C, N), lambda b, t: (b, 0, 0)),
            pl.BlockSpec((KNN, C, O), lambda b, t: (0, 0, 0)),
            pl.BlockSpec((1, O), lambda b, t: (0, 0)),
        ],
        out_specs=(
            [pl.BlockSpec((1, tr, O), lambda b, t: (b, t, 0))] * KNN
            + [pl.BlockSpec((1, tr, 1), lambda b, t: (b, t, 0))] * KNN
        ),
        out_shape=(
            [jax.ShapeDtypeStruct((B, N, O), jnp.float32)] * KNN
            + [jax.ShapeDtypeStruct((B, N, 1), jnp.int32)] * KNN
        ),
    )(x, x, wt, bias2)


def _make_sc_gather_sum(bn, o, ch):
    info = plsc.get_sparse_core_info()
    nc, ns = info.num_cores, info.num_subcores
    nw = nc * ns
    rows_per_w = bn // nw
    n_chunks = rows_per_w // ch
    mesh = plsc.VectorSubcoreMesh(core_axis_name="c", subcore_axis_name="s")

    @functools.partial(
        pl.kernel,
        out_type=jax.ShapeDtypeStruct((bn, o), jnp.float32),
        mesh=mesh,
        scratch_types=(
            [pltpu.VMEM((rows_per_w,), jnp.int32) for _ in range(KNN)]
            + [pltpu.VMEM((ch, o), jnp.float32) for _ in range(2 * KNN)]
            + [pltpu.VMEM((ch, o), jnp.float32) for _ in range(2)]
            + [pltpu.SemaphoreType.DMA for _ in range(2)]
            + [pltpu.SemaphoreType.DMA for _ in range(2)]
        ),
    )
    def sc_gather_sum(y0, y1, y2, y3, i0, i1, i2, i3, out,
                      ib0, ib1, ib2, ib3,
                      ga0, ga1, ga2, ga3, gb0, gb1, gb2, gb3,
                      oba, obb, sg0, sg1, so0, so1):
        wid = lax.axis_index("s") * nc + lax.axis_index("c")
        base0 = wid * rows_per_w
        ys = (y0, y1, y2, y3)
        idx_hbm = (i0, i1, i2, i3)
        ibs = (ib0, ib1, ib2, ib3)
        gsets = ((ga0, ga1, ga2, ga3), (gb0, gb1, gb2, gb3))
        obufs = (oba, obb)
        gsems = (sg0, sg1)
        osems = (so0, so1)

        for k in range(KNN):
            pltpu.sync_copy(idx_hbm[k].at[pl.ds(base0, rows_per_w)], ibs[k])

        def fire(g):
            p = g % 2
            return [pltpu.async_copy(
                        ys[k].at[ibs[k].at[pl.ds(g * ch, ch)]],
                        gsets[p][k], gsems[p])
                    for k in range(KNN)]

        gath = {0: fire(0)}
        ocopies = {}
        for g in range(n_chunks):
            p = g % 2
            if g + 1 < n_chunks:
                gath[g + 1] = fire(g + 1)
            for c in gath.pop(g):
                c.wait()
            if g >= 2:
                for c in ocopies.pop(g - 2):
                    c.wait()
            gbufs = gsets[p]
            ob = obufs[p]

            def row_body(j, carry):
                for c16 in range(o // 16):
                    sl = pl.ds(c16 * 16, 16)
                    ob[j, sl] = (gbufs[0][j, sl] + gbufs[1][j, sl]
                                 + gbufs[2][j, sl] + gbufs[3][j, sl])
                return carry

            lax.fori_loop(0, ch, row_body, 0)
            ocopies[g] = [pltpu.async_copy(
                ob, out.at[pl.ds(base0 + g * ch, ch)], osems[p])]
        for g in list(ocopies):
            for c in ocopies.pop(g):
                c.wait()

    return sc_gather_sum


def kernel(x, W, b):
    B, C, N = x.shape
    O = W.shape[0]
    wt = W.transpose(2, 1, 0)
    bias2 = b.reshape(1, O)

    outs = _tc_stage(x, wt, bias2, tr=1024)
    ys = [o.reshape(B * N, O) for o in outs[:KNN]]
    idxs = [o.reshape(B * N) for o in outs[KNN:]]

    sc = _make_sc_gather_sum(B * N, O, ch=32)
    out_t = sc(*ys, *idxs)
    return out_t.reshape(B, N, O).transpose(0, 2, 1)


# speedup vs baseline: 1.2826x; 1.0454x over previous
import functools

import jax
import jax.numpy as jnp
from jax import lax
from jax.experimental import pallas as pl
from jax.experimental.pallas import tpu as pltpu
from jax.experimental.pallas import tpu_sc as plsc

KNN = 4


def _knn_y_kernel(xf_ref, xb_ref, wt_ref, bias_ref,
                  y0, y1, y2, y3, i0, i1, i2, i3):
    xf = xf_ref[0]
    a = xb_ref[0]
    n = xf.shape[0]
    tr = a.shape[0]

    dot = lax.dot_general(xf, a, (((1,), (1,)), ((), ())),
                          preferred_element_type=jnp.float32)
    sq_f = jnp.sum(xf * xf, axis=1, keepdims=True)
    sq_b = jnp.swapaxes(jnp.sum(a * a, axis=1, keepdims=True), 0, 1)
    d = jnp.sqrt(jnp.maximum(sq_f + sq_b - 2.0 * dot, 0.0))

    iota = lax.broadcasted_iota(jnp.int32, (n, tr), 0)
    gbase = pl.program_id(0) * n
    idx_refs = (i0, i1, i2, i3)
    y_refs = (y0, y1, y2, y3)
    for k in range(KNN):
        m = jnp.min(d, axis=0, keepdims=True)
        ik = jnp.min(jnp.where(d == m, iota, n), axis=0, keepdims=True)
        idx_refs[k][0] = ik + gbase
        if k + 1 < KNN:
            d = jnp.where(iota == ik, jnp.inf, d)
        yk = lax.dot_general(a, wt_ref[k], (((1,), (0,)), ((), ())),
                             preferred_element_type=jnp.float32)
        if k == 0:
            yk = yk + bias_ref[...]
        y_refs[k][0] = yk


def _tc_stage(xt, wt, bias2, tr):
    B, N, C = xt.shape
    O = wt.shape[2]
    return pl.pallas_call(
        _knn_y_kernel,
        grid=(B, N // tr),
        in_specs=[
            pl.BlockSpec((1, N, C), lambda b, t: (b, 0, 0)),
            pl.BlockSpec((1, tr, C), lambda b, t: (b, t, 0)),
            pl.BlockSpec((KNN, C, O), lambda b, t: (0, 0, 0)),
            pl.BlockSpec((1, O), lambda b, t: (0, 0)),
        ],
        out_specs=(
            [pl.BlockSpec((1, tr, O), lambda b, t: (b, t, 0))] * KNN
            + [pl.BlockSpec((1, 1, tr), lambda b, t: (b, 0, t))] * KNN
        ),
        out_shape=(
            [jax.ShapeDtypeStruct((B, N, O), jnp.float32)] * KNN
            + [jax.ShapeDtypeStruct((B, 1, N), jnp.int32)] * KNN
        ),
    )(xt, xt, wt, bias2)


def _make_sc_gather_sum(bn, o, ch):
    info = plsc.get_sparse_core_info()
    nc, ns = info.num_cores, info.num_subcores
    nw = nc * ns
    rows_per_w = bn // nw
    n_chunks = rows_per_w // ch
    mesh = plsc.VectorSubcoreMesh(core_axis_name="c", subcore_axis_name="s")

    @functools.partial(
        pl.kernel,
        out_type=jax.ShapeDtypeStruct((bn, o), jnp.float32),
        mesh=mesh,
        scratch_types=(
            [pltpu.VMEM((rows_per_w,), jnp.int32) for _ in range(KNN)]
            + [pltpu.VMEM((ch, o), jnp.float32) for _ in range(2 * KNN)]
            + [pltpu.VMEM((ch, o), jnp.float32) for _ in range(2)]
            + [pltpu.SemaphoreType.DMA for _ in range(2)]
            + [pltpu.SemaphoreType.DMA for _ in range(2)]
        ),
    )
    def sc_gather_sum(y0, y1, y2, y3, i0, i1, i2, i3, out,
                      ib0, ib1, ib2, ib3,
                      ga0, ga1, ga2, ga3, gb0, gb1, gb2, gb3,
                      oba, obb, sg0, sg1, so0, so1):
        wid = lax.axis_index("s") * nc + lax.axis_index("c")
        base0 = wid * rows_per_w
        ys = (y0, y1, y2, y3)
        idx_hbm = (i0, i1, i2, i3)
        ibs = (ib0, ib1, ib2, ib3)
        gsets = ((ga0, ga1, ga2, ga3), (gb0, gb1, gb2, gb3))
        obufs = (oba, obb)
        gsems = (sg0, sg1)
        osems = (so0, so1)

        for k in range(KNN):
            pltpu.sync_copy(idx_hbm[k].at[pl.ds(base0, rows_per_w)], ibs[k])

        def fire(g):
            p = g % 2
            return [pltpu.async_copy(
                        ys[k].at[ibs[k].at[pl.ds(g * ch, ch)]],
                        gsets[p][k], gsems[p])
                    for k in range(KNN)]

        gath = {0: fire(0)}
        ocopies = {}
        for g in range(n_chunks):
            p = g % 2
            if g + 1 < n_chunks:
                gath[g + 1] = fire(g + 1)
            for c in gath.pop(g):
                c.wait()
            if g >= 2:
                for c in ocopies.pop(g - 2):
                    c.wait()
            gbufs = gsets[p]
            ob = obufs[p]

            def row_body(j, carry):
                for c16 in range(o // 16):
                    sl = pl.ds(c16 * 16, 16)
                    ob[j, sl] = (gbufs[0][j, sl] + gbufs[1][j, sl]
                                 + gbufs[2][j, sl] + gbufs[3][j, sl])
                return carry

            lax.fori_loop(0, ch, row_body, 0)
            ocopies[g] = [pltpu.async_copy(
                ob, out.at[pl.ds(base0 + g * ch, ch)], osems[p])]
        for g in list(ocopies):
            for c in ocopies.pop(g):
                c.wait()

    return sc_gather_sum


def kernel(x, W, b):
    B, C, N = x.shape
    O = W.shape[0]
    xt = x.transpose(0, 2, 1)
    wt = W.transpose(2, 1, 0)
    bias2 = b.reshape(1, O)

    outs = _tc_stage(xt, wt, bias2, tr=1024)
    ys = [o.reshape(B * N, O) for o in outs[:KNN]]
    idxs = [o.reshape(B * N) for o in outs[KNN:]]

    sc = _make_sc_gather_sum(B * N, O, ch=32)
    out_t = sc(*ys, *idxs)
    return out_t.reshape(B, N, O).transpose(0, 2, 1)


# speedup vs baseline: 1.3450x; 1.0487x over previous
import functools

import jax
import jax.numpy as jnp
from jax import lax
from jax.experimental import pallas as pl
from jax.experimental.pallas import tpu as pltpu
from jax.experimental.pallas import tpu_sc as plsc

KNN = 4


def _knn_y_kernel(xf_ref, xb_ref, wt_ref, bias_ref,
                  y0, y1, y2, y3, i0, i1, i2, i3):
    xf = xf_ref[0]
    a = xb_ref[0]
    n = xf.shape[0]
    tr = a.shape[0]

    dot = lax.dot_general(xf, a, (((1,), (1,)), ((), ())),
                          preferred_element_type=jnp.float32)
    sq_f = jnp.sum(xf * xf, axis=1, keepdims=True)
    sq_b = jnp.swapaxes(jnp.sum(a * a, axis=1, keepdims=True), 0, 1)
    d = jnp.sqrt(jnp.maximum(sq_f + sq_b - 2.0 * dot, 0.0))

    iota = lax.broadcasted_iota(jnp.int32, (n, tr), 0).astype(jnp.float32)
    nf = jnp.float32(n)
    gbase = pl.program_id(0) * n
    idx_refs = (i0, i1, i2, i3)
    y_refs = (y0, y1, y2, y3)
    for k in range(KNN):
        m = jnp.min(d, axis=0, keepdims=True)
        ikf = jnp.min(jnp.where(d == m, iota, nf), axis=0, keepdims=True)
        idx_refs[k][0] = ikf.astype(jnp.int32) + gbase
        if k + 1 < KNN:
            d = jnp.where(iota == ikf, jnp.inf, d)
        yk = lax.dot_general(a, wt_ref[k], (((1,), (0,)), ((), ())),
                             preferred_element_type=jnp.float32)
        if k == 0:
            yk = yk + bias_ref[...]
        y_refs[k][0] = yk


def _tc_stage(xt, wt, bias2, tr):
    B, N, C = xt.shape
    O = wt.shape[2]
    return pl.pallas_call(
        _knn_y_kernel,
        grid=(B, N // tr),
        in_specs=[
            pl.BlockSpec((1, N, C), lambda b, t: (b, 0, 0)),
            pl.BlockSpec((1, tr, C), lambda b, t: (b, t, 0)),
            pl.BlockSpec((KNN, C, O), lambda b, t: (0, 0, 0)),
            pl.BlockSpec((1, O), lambda b, t: (0, 0)),
        ],
        out_specs=(
            [pl.BlockSpec((1, tr, O), lambda b, t: (b, t, 0))] * KNN
            + [pl.BlockSpec((1, 1, tr), lambda b, t: (b, 0, t))] * KNN
        ),
        out_shape=(
            [jax.ShapeDtypeStruct((B, N, O), jnp.float32)] * KNN
            + [jax.ShapeDtypeStruct((B, 1, N), jnp.int32)] * KNN
        ),
    )(xt, xt, wt, bias2)


def _make_sc_gather_sum(bn, o, ch):
    info = plsc.get_sparse_core_info()
    nc, ns = info.num_cores, info.num_subcores
    nw = nc * ns
    rows_per_w = bn // nw
    n_chunks = rows_per_w // ch
    mesh = plsc.VectorSubcoreMesh(core_axis_name="c", subcore_axis_name="s")

    @functools.partial(
        pl.kernel,
        out_type=jax.ShapeDtypeStruct((bn, o), jnp.float32),
        mesh=mesh,
        scratch_types=(
            [pltpu.VMEM((rows_per_w,), jnp.int32) for _ in range(KNN)]
            + [pltpu.VMEM((ch, o), jnp.float32) for _ in range(2 * KNN)]
            + [pltpu.VMEM((ch, o), jnp.float32) for _ in range(2)]
            + [pltpu.SemaphoreType.DMA for _ in range(2)]
            + [pltpu.SemaphoreType.DMA for _ in range(2)]
        ),
    )
    def sc_gather_sum(y0, y1, y2, y3, i0, i1, i2, i3, out,
                      ib0, ib1, ib2, ib3,
                      ga0, ga1, ga2, ga3, gb0, gb1, gb2, gb3,
                      oba, obb, sg0, sg1, so0, so1):
        wid = lax.axis_index("s") * nc + lax.axis_index("c")
        base0 = wid * rows_per_w
        ys = (y0, y1, y2, y3)
        idx_hbm = (i0, i1, i2, i3)
        ibs = (ib0, ib1, ib2, ib3)
        gsets = ((ga0, ga1, ga2, ga3), (gb0, gb1, gb2, gb3))
        obufs = (oba, obb)
        gsems = (sg0, sg1)
        osems = (so0, so1)

        for k in range(KNN):
            pltpu.sync_copy(idx_hbm[k].at[pl.ds(base0, rows_per_w)], ibs[k])

        def fire(g):
            p = g % 2
            return [pltpu.async_copy(
                        ys[k].at[ibs[k].at[pl.ds(g * ch, ch)]],
                        gsets[p][k], gsems[p])
                    for k in range(KNN)]

        gath = {0: fire(0)}
        ocopies = {}
        for g in range(n_chunks):
            p = g % 2
            if g + 1 < n_chunks:
                gath[g + 1] = fire(g + 1)
            for c in gath.pop(g):
                c.wait()
            if g >= 2:
                for c in ocopies.pop(g - 2):
                    c.wait()
            gbufs = gsets[p]
            ob = obufs[p]

            def row_body(j, carry):
                for c16 in range(o // 16):
                    sl = pl.ds(c16 * 16, 16)
                    ob[j, sl] = (gbufs[0][j, sl] + gbufs[1][j, sl]
                                 + gbufs[2][j, sl] + gbufs[3][j, sl])
                return carry

            lax.fori_loop(0, ch, row_body, 0)
            ocopies[g] = [pltpu.async_copy(
                ob, out.at[pl.ds(base0 + g * ch, ch)], osems[p])]
        for g in list(ocopies):
            for c in ocopies.pop(g):
                c.wait()

    return sc_gather_sum


def kernel(x, W, b):
    B, C, N = x.shape
    O = W.shape[0]
    xt = x.transpose(0, 2, 1)
    wt = W.transpose(2, 1, 0)
    bias2 = b.reshape(1, O)

    outs = _tc_stage(xt, wt, bias2, tr=1024)
    ys = [o.reshape(B * N, O) for o in outs[:KNN]]
    idxs = [o.reshape(B * N) for o in outs[KNN:]]

    sc = _make_sc_gather_sum(B * N, O, ch=32)
    out_t = sc(*ys, *idxs)
    return out_t.reshape(B, N, O).transpose(0, 2, 1)


# speedup vs baseline: 1.3888x; 1.0325x over previous
import functools

import jax
import jax.numpy as jnp
from jax import lax
from jax.experimental import pallas as pl
from jax.experimental.pallas import tpu as pltpu
from jax.experimental.pallas import tpu_sc as plsc

KNN = 4


def _knn_y_kernel(xf_ref, xb_ref, wt_ref, bias_ref,
                  y0, y1, y2, y3, i0, i1, i2, i3):
    xf = jnp.swapaxes(xf_ref[0], 0, 1)
    a = jnp.swapaxes(xb_ref[0], 0, 1)
    n = xf.shape[0]
    tr = a.shape[0]

    dot = lax.dot_general(xf, a, (((1,), (1,)), ((), ())),
                          preferred_element_type=jnp.float32)
    sq_f = jnp.sum(xf * xf, axis=1, keepdims=True)
    sq_b = jnp.swapaxes(jnp.sum(a * a, axis=1, keepdims=True), 0, 1)
    d = jnp.sqrt(jnp.maximum(sq_f + sq_b - 2.0 * dot, 0.0))

    iota = lax.broadcasted_iota(jnp.int32, (n, tr), 0).astype(jnp.float32)
    nf = jnp.float32(n)
    gbase = pl.program_id(0) * n
    idx_refs = (i0, i1, i2, i3)
    y_refs = (y0, y1, y2, y3)
    for k in range(KNN):
        m = jnp.min(d, axis=0, keepdims=True)
        ikf = jnp.min(jnp.where(d == m, iota, nf), axis=0, keepdims=True)
        idx_refs[k][0] = ikf.astype(jnp.int32) + gbase
        if k + 1 < KNN:
            d = jnp.where(iota == ikf, jnp.inf, d)
        yk = lax.dot_general(a, wt_ref[k], (((1,), (0,)), ((), ())),
                             preferred_element_type=jnp.float32)
        if k == 0:
            yk = yk + bias_ref[...]
        y_refs[k][0] = yk


def _tc_stage(x, wt, bias2, tr):
    B, C, N = x.shape
    O = wt.shape[2]
    return pl.pallas_call(
        _knn_y_kernel,
        grid=(B, N // tr),
        in_specs=[
            pl.BlockSpec((1, C, N), lambda b, t: (b, 0, 0)),
            pl.BlockSpec((1, C, tr), lambda b, t: (b, 0, t)),
            pl.BlockSpec((KNN, C, O), lambda b, t: (0, 0, 0)),
            pl.BlockSpec((1, O), lambda b, t: (0, 0)),
        ],
        out_specs=(
            [pl.BlockSpec((1, tr, O), lambda b, t: (b, t, 0))] * KNN
            + [pl.BlockSpec((1, 1, tr), lambda b, t: (b, 0, t))] * KNN
        ),
        out_shape=(
            [jax.ShapeDtypeStruct((B, N, O), jnp.float32)] * KNN
            + [jax.ShapeDtypeStruct((B, 1, N), jnp.int32)] * KNN
        ),
    )(x, x, wt, bias2)


def _make_sc_gather_sum(bn, o, ch):
    info = plsc.get_sparse_core_info()
    nc, ns = info.num_cores, info.num_subcores
    nw = nc * ns
    rows_per_w = bn // nw
    n_chunks = rows_per_w // ch
    mesh = plsc.VectorSubcoreMesh(core_axis_name="c", subcore_axis_name="s")

    @functools.partial(
        pl.kernel,
        out_type=jax.ShapeDtypeStruct((bn, o), jnp.float32),
        mesh=mesh,
        scratch_types=(
            [pltpu.VMEM((rows_per_w,), jnp.int32) for _ in range(KNN)]
            + [pltpu.VMEM((ch, o), jnp.float32) for _ in range(2 * KNN)]
            + [pltpu.VMEM((ch, o), jnp.float32) for _ in range(2)]
            + [pltpu.SemaphoreType.DMA for _ in range(2)]
            + [pltpu.SemaphoreType.DMA for _ in range(2)]
        ),
    )
    def sc_gather_sum(y0, y1, y2, y3, i0, i1, i2, i3, out,
                      ib0, ib1, ib2, ib3,
                      ga0, ga1, ga2, ga3, gb0, gb1, gb2, gb3,
                      oba, obb, sg0, sg1, so0, so1):
        wid = lax.axis_index("s") * nc + lax.axis_index("c")
        base0 = wid * rows_per_w
        ys = (y0, y1, y2, y3)
        idx_hbm = (i0, i1, i2, i3)
        ibs = (ib0, ib1, ib2, ib3)
        gsets = ((ga0, ga1, ga2, ga3), (gb0, gb1, gb2, gb3))
        obufs = (oba, obb)
        gsems = (sg0, sg1)
        osems = (so0, so1)

        for k in range(KNN):
            pltpu.sync_copy(idx_hbm[k].at[pl.ds(base0, rows_per_w)], ibs[k])

        def fire(g):
            p = g % 2
            return [pltpu.async_copy(
                        ys[k].at[ibs[k].at[pl.ds(g * ch, ch)]],
                        gsets[p][k], gsems[p])
                    for k in range(KNN)]

        gath = {0: fire(0)}
        ocopies = {}
        for g in range(n_chunks):
            p = g % 2
            if g + 1 < n_chunks:
                gath[g + 1] = fire(g + 1)
            for c in gath.pop(g):
                c.wait()
            if g >= 2:
                for c in ocopies.pop(g - 2):
                    c.wait()
            gbufs = gsets[p]
            ob = obufs[p]

            def row_body(j, carry):
                for c16 in range(o // 16):
                    sl = pl.ds(c16 * 16, 16)
                    ob[j, sl] = (gbufs[0][j, sl] + gbufs[1][j, sl]
                                 + gbufs[2][j, sl] + gbufs[3][j, sl])
                return carry

            lax.fori_loop(0, ch, row_body, 0)
            ocopies[g] = [pltpu.async_copy(
                ob, out.at[pl.ds(base0 + g * ch, ch)], osems[p])]
        for g in list(ocopies):
            for c in ocopies.pop(g):
                c.wait()

    return sc_gather_sum


def kernel(x, W, b):
    B, C, N = x.shape
    O = W.shape[0]
    wt = W.transpose(2, 1, 0)
    bias2 = b.reshape(1, O)

    outs = _tc_stage(x, wt, bias2, tr=1024)
    ys = [o.reshape(B * N, O) for o in outs[:KNN]]
    idxs = [o.reshape(B * N) for o in outs[KNN:]]

    sc = _make_sc_gather_sum(B * N, O, ch=32)
    out_t = sc(*ys, *idxs)
    return out_t.reshape(B, N, O).transpose(0, 2, 1)


# speedup vs baseline: 1.4477x; 1.0425x over previous
import functools

import jax
import jax.numpy as jnp
from jax import lax
from jax.experimental import pallas as pl
from jax.experimental.pallas import tpu as pltpu
from jax.experimental.pallas import tpu_sc as plsc

KNN = 4


def _knn_y_kernel(xf_ref, xb_ref, wt_ref, bias_ref,
                  y0, y1, y2, y3, i0, i1, i2, i3):
    xf = jnp.swapaxes(xf_ref[0], 0, 1)
    a = jnp.swapaxes(xb_ref[0], 0, 1)
    n = xf.shape[0]
    tr = a.shape[0]

    dot = lax.dot_general(xf, a, (((1,), (1,)), ((), ())),
                          preferred_element_type=jnp.float32)
    sq_f = jnp.sum(xf * xf, axis=1, keepdims=True)
    sq_b = jnp.swapaxes(jnp.sum(a * a, axis=1, keepdims=True), 0, 1)
    d = jnp.sqrt(jnp.maximum(sq_f + sq_b - 2.0 * dot, 0.0))

    iota = lax.broadcasted_iota(jnp.int32, (n, tr), 0).astype(jnp.float32)
    nf = jnp.float32(n)
    gbase = pl.program_id(0) * n
    idx_refs = (i0, i1, i2, i3)
    y_refs = (y0, y1, y2, y3)
    del nf
    for k in range(KNN):
        ik = jnp.argmin(d, axis=0).reshape(1, tr)
        idx_refs[k][0] = ik + gbase
        if k + 1 < KNN:
            d = jnp.where(iota == ik.astype(jnp.float32), jnp.inf, d)
        yk = lax.dot_general(a, wt_ref[k], (((1,), (0,)), ((), ())),
                             preferred_element_type=jnp.float32)
        if k == 0:
            yk = yk + bias_ref[...]
        y_refs[k][0] = yk


def _tc_stage(x, wt, bias2, tr):
    B, C, N = x.shape
    O = wt.shape[2]
    return pl.pallas_call(
        _knn_y_kernel,
        grid=(B, N // tr),
        in_specs=[
            pl.BlockSpec((1, C, N), lambda b, t: (b, 0, 0)),
            pl.BlockSpec((1, C, tr), lambda b, t: (b, 0, t)),
            pl.BlockSpec((KNN, C, O), lambda b, t: (0, 0, 0)),
            pl.BlockSpec((1, O), lambda b, t: (0, 0)),
        ],
        out_specs=(
            [pl.BlockSpec((1, tr, O), lambda b, t: (b, t, 0))] * KNN
            + [pl.BlockSpec((1, 1, tr), lambda b, t: (b, 0, t))] * KNN
        ),
        out_shape=(
            [jax.ShapeDtypeStruct((B, N, O), jnp.float32)] * KNN
            + [jax.ShapeDtypeStruct((B, 1, N), jnp.int32)] * KNN
        ),
    )(x, x, wt, bias2)


def _make_sc_gather_sum(bn, o, ch):
    info = plsc.get_sparse_core_info()
    nc, ns = info.num_cores, info.num_subcores
    nw = nc * ns
    rows_per_w = bn // nw
    n_chunks = rows_per_w // ch
    mesh = plsc.VectorSubcoreMesh(core_axis_name="c", subcore_axis_name="s")

    @functools.partial(
        pl.kernel,
        out_type=jax.ShapeDtypeStruct((bn, o), jnp.float32),
        mesh=mesh,
        scratch_types=(
            [pltpu.VMEM((rows_per_w,), jnp.int32) for _ in range(KNN)]
            + [pltpu.VMEM((ch, o), jnp.float32) for _ in range(2 * KNN)]
            + [pltpu.VMEM((ch, o), jnp.float32) for _ in range(2)]
            + [pltpu.SemaphoreType.DMA for _ in range(2)]
            + [pltpu.SemaphoreType.DMA for _ in range(2)]
        ),
    )
    def sc_gather_sum(y0, y1, y2, y3, i0, i1, i2, i3, out,
                      ib0, ib1, ib2, ib3,
                      ga0, ga1, ga2, ga3, gb0, gb1, gb2, gb3,
                      oba, obb, sg0, sg1, so0, so1):
        wid = lax.axis_index("s") * nc + lax.axis_index("c")
        base0 = wid * rows_per_w
        ys = (y0, y1, y2, y3)
        idx_hbm = (i0, i1, i2, i3)
        ibs = (ib0, ib1, ib2, ib3)
        gsets = ((ga0, ga1, ga2, ga3), (gb0, gb1, gb2, gb3))
        obufs = (oba, obb)
        gsems = (sg0, sg1)
        osems = (so0, so1)

        for k in range(KNN):
            pltpu.sync_copy(idx_hbm[k].at[pl.ds(base0, rows_per_w)], ibs[k])

        def fire(g):
            p = g % 2
            return [pltpu.async_copy(
                        ys[k].at[ibs[k].at[pl.ds(g * ch, ch)]],
                        gsets[p][k], gsems[p])
                    for k in range(KNN)]

        gath = {0: fire(0)}
        ocopies = {}
        for g in range(n_chunks):
            p = g % 2
            if g + 1 < n_chunks:
                gath[g + 1] = fire(g + 1)
            for c in gath.pop(g):
                c.wait()
            if g >= 2:
                for c in ocopies.pop(g - 2):
                    c.wait()
            gbufs = gsets[p]
            ob = obufs[p]

            def row_body(j, carry):
                for c16 in range(o // 16):
                    sl = pl.ds(c16 * 16, 16)
                    ob[j, sl] = (gbufs[0][j, sl] + gbufs[1][j, sl]
                                 + gbufs[2][j, sl] + gbufs[3][j, sl])
                return carry

            lax.fori_loop(0, ch, row_body, 0)
            ocopies[g] = [pltpu.async_copy(
                ob, out.at[pl.ds(base0 + g * ch, ch)], osems[p])]
        for g in list(ocopies):
            for c in ocopies.pop(g):
                c.wait()

    return sc_gather_sum


def kernel(x, W, b):
    B, C, N = x.shape
    O = W.shape[0]
    wt = W.transpose(2, 1, 0)
    bias2 = b.reshape(1, O)

    outs = _tc_stage(x, wt, bias2, tr=1024)
    ys = [o.reshape(B * N, O) for o in outs[:KNN]]
    idxs = [o.reshape(B * N) for o in outs[KNN:]]

    sc = _make_sc_gather_sum(B * N, O, ch=32)
    out_t = sc(*ys, *idxs)
    return out_t.reshape(B, N, O).transpose(0, 2, 1)


# speedup vs baseline: 1.4541x; 1.0044x over previous
import functools

import jax
import jax.numpy as jnp
from jax import lax
from jax.experimental import pallas as pl
from jax.experimental.pallas import tpu as pltpu
from jax.experimental.pallas import tpu_sc as plsc

KNN = 4


def _knn_y_kernel(xf_ref, xb_ref, wt_ref, bias_ref,
                  y0, y1, y2, y3, i0, i1, i2, i3):
    xf = jnp.swapaxes(xf_ref[0], 0, 1)
    a = jnp.swapaxes(xb_ref[0], 0, 1)
    n = xf.shape[0]
    tr = a.shape[0]

    dot = lax.dot_general(xf, a, (((1,), (1,)), ((), ())),
                          preferred_element_type=jnp.float32)
    sq_f = jnp.sum(xf * xf, axis=1, keepdims=True)
    sq_b = jnp.swapaxes(jnp.sum(a * a, axis=1, keepdims=True), 0, 1)
    d = jnp.sqrt(jnp.maximum(sq_f + sq_b - 2.0 * dot, 0.0))

    iota = lax.broadcasted_iota(jnp.int32, (n, tr), 0).astype(jnp.float32)
    nf = jnp.float32(n)
    gbase = pl.program_id(0) * n
    idx_refs = (i0, i1, i2, i3)
    y_refs = (y0, y1, y2, y3)
    del nf
    for k in range(KNN):
        ik = jnp.argmin(d, axis=0).reshape(1, tr)
        idx_refs[k][0] = ik + gbase
        if k + 1 < KNN:
            d = jnp.where(iota == ik.astype(jnp.float32), jnp.inf, d)
        yk = lax.dot_general(a, wt_ref[k], (((1,), (0,)), ((), ())),
                             preferred_element_type=jnp.float32)
        if k == 0:
            yk = yk + bias_ref[...]
        y_refs[k][0] = yk


def _tc_stage(x, wt, bias2, tr):
    B, C, N = x.shape
    O = wt.shape[2]
    return pl.pallas_call(
        _knn_y_kernel,
        grid=(B, N // tr),
        in_specs=[
            pl.BlockSpec((1, C, N), lambda b, t: (b, 0, 0)),
            pl.BlockSpec((1, C, tr), lambda b, t: (b, 0, t)),
            pl.BlockSpec((KNN, C, O), lambda b, t: (0, 0, 0)),
            pl.BlockSpec((1, O), lambda b, t: (0, 0)),
        ],
        out_specs=(
            [pl.BlockSpec((1, tr, O), lambda b, t: (b, t, 0))] * KNN
            + [pl.BlockSpec((1, 1, tr), lambda b, t: (b, 0, t))] * KNN
        ),
        out_shape=(
            [jax.ShapeDtypeStruct((B, N, O), jnp.float32)] * KNN
            + [jax.ShapeDtypeStruct((B, 1, N), jnp.int32)] * KNN
        ),
    )(x, x, wt, bias2)


def _make_sc_gather_sum(bn, o, ch):
    info = plsc.get_sparse_core_info()
    nc, ns = info.num_cores, info.num_subcores
    nw = nc * ns
    rows_per_w = bn // nw
    n_chunks = rows_per_w // ch
    mesh = plsc.VectorSubcoreMesh(core_axis_name="c", subcore_axis_name="s")

    @functools.partial(
        pl.kernel,
        out_type=jax.ShapeDtypeStruct((bn, o), jnp.float32),
        mesh=mesh,
        scratch_types=(
            [pltpu.VMEM((rows_per_w,), jnp.int32) for _ in range(KNN)]
            + [pltpu.VMEM((ch, o), jnp.float32) for _ in range(3 * KNN)]
            + [pltpu.SemaphoreType.DMA for _ in range(3)]
            + [pltpu.SemaphoreType.DMA for _ in range(3)]
        ),
    )
    def sc_gather_sum(y0, y1, y2, y3, i0, i1, i2, i3, out,
                      ib0, ib1, ib2, ib3,
                      ga0, ga1, ga2, ga3, gb0, gb1, gb2, gb3,
                      gc0, gc1, gc2, gc3,
                      sg0, sg1, sg2, so0, so1, so2):
        wid = lax.axis_index("s") * nc + lax.axis_index("c")
        base0 = wid * rows_per_w
        ys = (y0, y1, y2, y3)
        idx_hbm = (i0, i1, i2, i3)
        ibs = (ib0, ib1, ib2, ib3)
        gsets = ((ga0, ga1, ga2, ga3), (gb0, gb1, gb2, gb3),
                 (gc0, gc1, gc2, gc3))
        gsems = (sg0, sg1, sg2)
        osems = (so0, so1, so2)
        nbuf = 3

        for k in range(KNN):
            pltpu.sync_copy(idx_hbm[k].at[pl.ds(base0, rows_per_w)], ibs[k])

        def fire(g):
            p = g % nbuf
            return [pltpu.async_copy(
                        ys[k].at[ibs[k].at[pl.ds(g * ch, ch)]],
                        gsets[p][k], gsems[p])
                    for k in range(KNN)]

        gath = {0: fire(0), 1: fire(1)}
        ocopies = {}
        for g in range(n_chunks):
            p = g % nbuf
            if g + 2 < n_chunks:
                if g - 1 in ocopies:
                    for c in ocopies.pop(g - 1):
                        c.wait()
                gath[g + 2] = fire(g + 2)
            for c in gath.pop(g):
                c.wait()
            gbufs = gsets[p]

            def row_body(j, carry):
                for c16 in range(o // 16):
                    sl = pl.ds(c16 * 16, 16)
                    gbufs[0][j, sl] = (gbufs[0][j, sl] + gbufs[1][j, sl]
                                       + gbufs[2][j, sl] + gbufs[3][j, sl])
                return carry

            lax.fori_loop(0, ch, row_body, 0)
            ocopies[g] = [pltpu.async_copy(
                gbufs[0], out.at[pl.ds(base0 + g * ch, ch)], osems[p])]
        for g in list(ocopies):
            for c in ocopies.pop(g):
                c.wait()

    return sc_gather_sum


def kernel(x, W, b):
    B, C, N = x.shape
    O = W.shape[0]
    wt = W.transpose(2, 1, 0)
    bias2 = b.reshape(1, O)

    outs = _tc_stage(x, wt, bias2, tr=1024)
    ys = [o.reshape(B * N, O) for o in outs[:KNN]]
    idxs = [o.reshape(B * N) for o in outs[KNN:]]

    sc = _make_sc_gather_sum(B * N, O, ch=32)
    out_t = sc(*ys, *idxs)
    return out_t.reshape(B, N, O).transpose(0, 2, 1)


# speedup vs baseline: 1.4578x; 1.0025x over previous
import functools

import jax
import jax.numpy as jnp
from jax import lax
from jax.experimental import pallas as pl
from jax.experimental.pallas import tpu as pltpu
from jax.experimental.pallas import tpu_sc as plsc

KNN = 4


def _knn_y_kernel(xf_ref, xb_ref, wt_ref, bias_ref,
                  y0, y1, y2, y3, i0, i1, i2, i3):
    xf = jnp.swapaxes(xf_ref[0], 0, 1)
    a = jnp.swapaxes(xb_ref[0], 0, 1)
    n = xf.shape[0]
    tr = a.shape[0]

    dot = lax.dot_general(xf, a, (((1,), (1,)), ((), ())),
                          preferred_element_type=jnp.float32)
    sq_f = jnp.sum(xf * xf, axis=1, keepdims=True)
    sq_b = jnp.swapaxes(jnp.sum(a * a, axis=1, keepdims=True), 0, 1)
    d = jnp.sqrt(jnp.maximum(sq_f + sq_b - 2.0 * dot, 0.0))

    iota = lax.broadcasted_iota(jnp.int32, (n, tr), 0)
    gbase = pl.program_id(0) * n
    idx_refs = (i0, i1, i2, i3)
    y_refs = (y0, y1, y2, y3)
    for k in range(KNN):
        ik = jnp.argmin(d, axis=0).reshape(1, tr)
        idx_refs[k][0] = ik + gbase
        if k + 1 < KNN:
            d = jnp.where(iota == ik, jnp.inf, d)
        yk = lax.dot_general(a, wt_ref[k], (((1,), (0,)), ((), ())),
                             preferred_element_type=jnp.float32)
        if k == 0:
            yk = yk + bias_ref[...]
        y_refs[k][0] = yk


def _tc_stage(x, wt, bias2, tr):
    B, C, N = x.shape
    O = wt.shape[2]
    return pl.pallas_call(
        _knn_y_kernel,
        grid=(B, N // tr),
        in_specs=[
            pl.BlockSpec((1, C, N), lambda b, t: (b, 0, 0)),
            pl.BlockSpec((1, C, tr), lambda b, t: (b, 0, t)),
            pl.BlockSpec((KNN, C, O), lambda b, t: (0, 0, 0)),
            pl.BlockSpec((1, O), lambda b, t: (0, 0)),
        ],
        out_specs=(
            [pl.BlockSpec((1, tr, O), lambda b, t: (b, t, 0))] * KNN
            + [pl.BlockSpec((1, 1, tr), lambda b, t: (b, 0, t))] * KNN
        ),
        out_shape=(
            [jax.ShapeDtypeStruct((B, N, O), jnp.float32)] * KNN
            + [jax.ShapeDtypeStruct((B, 1, N), jnp.int32)] * KNN
        ),
    )(x, x, wt, bias2)


def _make_sc_gather_sum(bn, o, ch):
    info = plsc.get_sparse_core_info()
    nc, ns = info.num_cores, info.num_subcores
    nw = nc * ns
    rows_per_w = bn // nw
    n_chunks = rows_per_w // ch
    mesh = plsc.VectorSubcoreMesh(core_axis_name="c", subcore_axis_name="s")

    @functools.partial(
        pl.kernel,
        out_type=jax.ShapeDtypeStruct((bn, o), jnp.float32),
        mesh=mesh,
        scratch_types=(
            [pltpu.VMEM((rows_per_w,), jnp.int32) for _ in range(KNN)]
            + [pltpu.VMEM((ch, o), jnp.float32) for _ in range(3 * KNN)]
            + [pltpu.SemaphoreType.DMA for _ in range(3)]
            + [pltpu.SemaphoreType.DMA for _ in range(3)]
        ),
    )
    def sc_gather_sum(y0, y1, y2, y3, i0, i1, i2, i3, out,
                      ib0, ib1, ib2, ib3,
                      ga0, ga1, ga2, ga3, gb0, gb1, gb2, gb3,
                      gc0, gc1, gc2, gc3,
                      sg0, sg1, sg2, so0, so1, so2):
        wid = lax.axis_index("s") * nc + lax.axis_index("c")
        base0 = wid * rows_per_w
        ys = (y0, y1, y2, y3)
        idx_hbm = (i0, i1, i2, i3)
        ibs = (ib0, ib1, ib2, ib3)
        gsets = ((ga0, ga1, ga2, ga3), (gb0, gb1, gb2, gb3),
                 (gc0, gc1, gc2, gc3))
        gsems = (sg0, sg1, sg2)
        osems = (so0, so1, so2)
        nbuf = 3

        for k in range(KNN):
            pltpu.sync_copy(idx_hbm[k].at[pl.ds(base0, rows_per_w)], ibs[k])

        def fire(g):
            p = g % nbuf
            return [pltpu.async_copy(
                        ys[k].at[ibs[k].at[pl.ds(g * ch, ch)]],
                        gsets[p][k], gsems[p])
                    for k in range(KNN)]

        gath = {0: fire(0), 1: fire(1)}
        ocopies = {}
        for g in range(n_chunks):
            p = g % nbuf
            if g + 2 < n_chunks:
                if g - 1 in ocopies:
                    for c in ocopies.pop(g - 1):
                        c.wait()
                gath[g + 2] = fire(g + 2)
            for c in gath.pop(g):
                c.wait()
            gbufs = gsets[p]

            def row_body(j, carry):
                for c16 in range(o // 16):
                    sl = pl.ds(c16 * 16, 16)
                    gbufs[0][j, sl] = (gbufs[0][j, sl] + gbufs[1][j, sl]
                                       + gbufs[2][j, sl] + gbufs[3][j, sl])
                return carry

            lax.fori_loop(0, ch, row_body, 0)
            ocopies[g] = [pltpu.async_copy(
                gbufs[0], out.at[pl.ds(base0 + g * ch, ch)], osems[p])]
        for g in list(ocopies):
            for c in ocopies.pop(g):
                c.wait()

    return sc_gather_sum


def kernel(x, W, b):
    B, C, N = x.shape
    O = W.shape[0]
    wt = W.transpose(2, 1, 0)
    bias2 = b.reshape(1, O)

    outs = _tc_stage(x, wt, bias2, tr=1024)
    ys = [o.reshape(B * N, O) for o in outs[:KNN]]
    idxs = [o.reshape(B * N) for o in outs[KNN:]]

    sc = _make_sc_gather_sum(B * N, O, ch=32)
    out_t = sc(*ys, *idxs)
    return out_t.reshape(B, N, O).transpose(0, 2, 1)
